# baseline (speedup 1.0000x reference)
import jax
import jax.numpy as jnp
from jax.experimental import pallas as pl

_BATCH = 16384
_DIMS = [(24, 1052), (1052, 526), (526, 256), (256, 128), (128, 64), (64, 4)]
_NLAYERS = len(_DIMS)
_EPS = 1e-5
_NCODES = 64


def _bdot(a, b):
    return jax.lax.dot_general(
        a.astype(jnp.bfloat16), b.astype(jnp.bfloat16),
        dimension_numbers=(((1,), (1,)), ((), ())),
        preferred_element_type=jnp.float32)


def _body(*refs):
    xt_ref = refs[0]
    e_refs = refs[1:4]
    w_refs = refs[4:4 + _NLAYERS]
    b_refs = refs[4 + _NLAYERS:4 + 2 * _NLAYERS]
    g_refs = refs[4 + 2 * _NLAYERS:3 + 3 * _NLAYERS]
    be_refs = refs[3 + 3 * _NLAYERS:2 + 4 * _NLAYERS]
    out_ref = refs[-1]

    xt = xt_ref[...]
    code = xt[0:1, :] * 16 + xt[1:2, :] * 4 + xt[2:3, :]
    sub = jax.lax.broadcasted_iota(jnp.int32, (_NCODES, _BATCH), 0)
    oht = (code == sub).astype(jnp.bfloat16)

    ones = jnp.ones((_BATCH, 1), jnp.bfloat16)
    counts = jnp.dot(oht, ones, preferred_element_type=jnp.float32)
    w = counts * (1.0 / _BATCH)

    row = jax.lax.broadcasted_iota(jnp.int32, (_NCODES, 4), 0)
    col = jax.lax.broadcasted_iota(jnp.int32, (_NCODES, 4), 1)
    parts = []
    for t, shift in enumerate((4, 2, 0)):
        sel = (jnp.right_shift(row, shift) & 3) == col
        parts.append(jnp.dot(sel.astype(jnp.bfloat16),
                             e_refs[t][...].astype(jnp.bfloat16),
                             preferred_element_type=jnp.float32))
    h = jnp.concatenate(parts, axis=1)

    for i in range(_NLAYERS):
        z = _bdot(h, w_refs[i][...]) + b_refs[i][...]
        if i < _NLAYERS - 1:
            r = jnp.maximum(z, 0.0)
            m = jnp.sum(w * r, axis=0, keepdims=True)
            d = r - m
            v = jnp.sum(w * (d * d), axis=0, keepdims=True)
            h = d * (g_refs[i][...] * jax.lax.rsqrt(v + _EPS)) + be_refs[i][...]
        else:
            h = z

    h_hi = h.astype(jnp.bfloat16).astype(jnp.float32)
    h_lo = h - h_hi
    hl = jnp.concatenate([h_hi, h_lo], axis=1)
    hlt = jnp.transpose(hl).astype(jnp.bfloat16)
    g8 = jnp.dot(hlt, oht, preferred_element_type=jnp.float32)
    out_ref[...] = g8[0:4, :] + g8[4:8, :]


def kernel(params, x):
    args = [x.T]
    args += [params[f"E{t}"] for t in range(3)]
    args += [params[f"W{i}"] for i in range(_NLAYERS)]
    args += [params[f"b{i}"].reshape(1, -1) for i in range(_NLAYERS)]
    args += [params[f"g{i}"].reshape(1, -1) for i in range(_NLAYERS - 1)]
    args += [params[f"be{i}"].reshape(1, -1) for i in range(_NLAYERS - 1)]
    out_t = pl.pallas_call(
        _body,
        out_shape=jax.ShapeDtypeStruct((4, _BATCH), jnp.float32),
    )(*args)
    return out_t.T
